# baseline (speedup 1.0000x reference)
import functools

import jax
import jax.numpy as jnp
from jax import lax
from jax.experimental import pallas as pl
from jax.experimental.pallas import tpu as pltpu
from jax.experimental.pallas import tpu_sc as plsc

B, K, D = 4096, 32, 64
NC, NS = 2, 16
NW = NC * NS
BPW = B // NW
GB = 4
GIDX = GB * K
NG = BPW // GB
DW = D // 2

_HI = -65536
_RND = 0x7FFF


def _word_to_f32(w):
    even = plsc.bitcast(w << 16, jnp.float32)
    odd = plsc.bitcast(w & _HI, jnp.float32)
    return even, odd


def _f32_to_word(even, odd):
    e = plsc.bitcast(even, jnp.int32)
    e = e + _RND + ((e >> 16) & 1)
    o = plsc.bitcast(odd, jnp.int32)
    o = o + _RND + ((o >> 16) & 1)
    return lax.shift_right_logical(e, 16) | (o & _HI)


def _body(idx_hbm, sco_hbm, w_hbm, out_hbm,
          idx_v, sco_v, rows_a, rows_b, out_v, sem_a, sem_b):
    wid = lax.axis_index("s") * NC + lax.axis_index("c")
    base = wid * BPW
    w32 = w_hbm

    pltpu.sync_copy(idx_hbm.at[wid], idx_v)
    pltpu.sync_copy(sco_hbm.at[pl.ds(base, BPW)], sco_v)

    def start(g, buf, sem):
        pltpu.async_copy(w32.at[idx_v.at[g]], buf, sem)

    def wait(g, buf, sem):
        pltpu.make_async_copy(w32.at[idx_v.at[g]], buf, sem).wait()

    def compute(g, rows):
        for i in range(GB):
            b = g * GB + i
            svecs = [sco_v[b, pl.ds(16 * j, 16)] for j in range(K // 16)]
            acc = [jnp.zeros((16,), jnp.float32) for _ in range(4)]
            for k in range(K):
                r = i * K + k
                wlo = rows[r, pl.ds(0, 16)]
                whi = rows[r, pl.ds(16, 16)]
                e0, o0 = _word_to_f32(wlo)
                e1, o1 = _word_to_f32(whi)
                s = svecs[k // 16][k % 16]
                sv = jnp.full((16,), s, jnp.float32)
                acc[0] = acc[0] + sv * e0
                acc[1] = acc[1] + sv * o0
                acc[2] = acc[2] + sv * e1
                acc[3] = acc[3] + sv * o1
            out_v[b, pl.ds(0, 16)] = _f32_to_word(acc[0], acc[1])
            out_v[b, pl.ds(16, 16)] = _f32_to_word(acc[2], acc[3])

    start(0, rows_a, sem_a)
    start(1, rows_b, sem_b)

    def loop_body(g2, carry):
        g = g2 * 2
        wait(g, rows_a, sem_a)
        compute(g, rows_a)

        @pl.when(g + 2 < NG)
        def _():
            start(g + 2, rows_a, sem_a)

        wait(g + 1, rows_b, sem_b)
        compute(g + 1, rows_b)

        @pl.when(g + 3 < NG)
        def _():
            start(g + 3, rows_b, sem_b)

        return carry

    lax.fori_loop(0, NG // 2, loop_body, 0)

    pltpu.sync_copy(out_v, out_hbm.at[pl.ds(base, BPW)])


_sc_call = functools.partial(
    pl.kernel,
    out_type=jax.ShapeDtypeStruct((B, DW), jnp.int32),
    mesh=plsc.VectorSubcoreMesh(
        core_axis_name="c", subcore_axis_name="s",
        num_cores=NC, num_subcores=NS),
    compiler_params=pltpu.CompilerParams(
        needs_layout_passes=False, use_tc_tiling_on_sc=False),
    scratch_types=[
        pltpu.VMEM((NG, GIDX), jnp.int32),
        pltpu.VMEM((BPW, K), jnp.float32),
        pltpu.VMEM((GIDX, DW), jnp.int32),
        pltpu.VMEM((GIDX, DW), jnp.int32),
        pltpu.VMEM((BPW, DW), jnp.int32),
        pltpu.SemaphoreType.DMA,
        pltpu.SemaphoreType.DMA,
    ],
)(_body)


def kernel(indices, scores, weight):
    idx3 = indices.reshape(NW, NG, GIDX)
    w32 = jax.lax.bitcast_convert_type(
        weight.reshape(weight.shape[0], DW, 2), jnp.int32)
    out32 = _sc_call(idx3, scores, w32)
    return jax.lax.bitcast_convert_type(out32, jnp.bfloat16).reshape(B, D)
